# baseline (speedup 1.0000x reference)
import jax
import jax.numpy as jnp
from jax.experimental import pallas as pl
from jax.experimental.pallas import tpu as pltpu


def _copy_body(tab_ref, out_ref, sem):
    s = out_ref.shape[0]
    copy = pltpu.make_async_copy(tab_ref.at[pl.ds(0, s)], out_ref, sem)
    copy.start()
    copy.wait()


def kernel(x, pos_table):
    s = x.shape[0]
    n, e = pos_table.shape
    tab3 = pos_table.reshape(n, 1, e)
    out = pl.pallas_call(
        _copy_body,
        in_specs=[pl.BlockSpec(memory_space=pl.ANY)],
        out_specs=pl.BlockSpec(memory_space=pl.ANY),
        out_shape=jax.ShapeDtypeStruct((s, 1, e), pos_table.dtype),
        scratch_shapes=[pltpu.SemaphoreType.DMA],
    )(tab3)
    return out


# speedup vs baseline: 1.0016x; 1.0016x over previous
import jax
import jax.numpy as jnp
from jax.experimental import pallas as pl
from jax.experimental.pallas import tpu as pltpu


_NCHUNK = 16


def _copy_body(tab_ref, out_ref, sems):
    s = out_ref.shape[0]
    chunk = s // _NCHUNK
    copies = []
    for i in range(_NCHUNK):
        c = pltpu.make_async_copy(
            tab_ref.at[pl.ds(i * chunk, chunk)],
            out_ref.at[pl.ds(i * chunk, chunk)],
            sems.at[i],
        )
        c.start()
        copies.append(c)
    for c in copies:
        c.wait()


def kernel(x, pos_table):
    s = x.shape[0]
    n, e = pos_table.shape
    tab3 = pos_table.reshape(n, 1, e)
    out = pl.pallas_call(
        _copy_body,
        in_specs=[pl.BlockSpec(memory_space=pl.ANY)],
        out_specs=pl.BlockSpec(memory_space=pl.ANY),
        out_shape=jax.ShapeDtypeStruct((s, 1, e), pos_table.dtype),
        scratch_shapes=[pltpu.SemaphoreType.DMA((_NCHUNK,))],
    )(tab3)
    return out


# speedup vs baseline: 12.3371x; 12.3170x over previous
import jax
import jax.numpy as jnp
from jax.experimental import pallas as pl
from jax.experimental.pallas import tpu as pltpu


_BLK = 512


def _copy_body(tab_ref, out_ref):
    out_ref[...] = tab_ref[...]


def kernel(x, pos_table):
    s = x.shape[0]
    n, e = pos_table.shape
    out = pl.pallas_call(
        _copy_body,
        grid=(s // _BLK,),
        in_specs=[pl.BlockSpec((_BLK, e), lambda i: (i, 0))],
        out_specs=pl.BlockSpec((_BLK, e), lambda i: (i, 0)),
        out_shape=jax.ShapeDtypeStruct((s, e), pos_table.dtype),
    )(pos_table)
    return out.reshape(s, 1, e)


# speedup vs baseline: 12.7000x; 1.0294x over previous
import jax
import jax.numpy as jnp
from jax.experimental import pallas as pl
from jax.experimental.pallas import tpu as pltpu


_BLK = 1024


def _copy_body(tab_ref, out_ref):
    out_ref[...] = tab_ref[...]


def kernel(x, pos_table):
    s = x.shape[0]
    n, e = pos_table.shape
    out = pl.pallas_call(
        _copy_body,
        grid=(s // _BLK,),
        in_specs=[pl.BlockSpec((_BLK, e), lambda i: (i, 0))],
        out_specs=pl.BlockSpec((_BLK, e), lambda i: (i, 0)),
        out_shape=jax.ShapeDtypeStruct((s, e), pos_table.dtype),
    )(pos_table)
    return out.reshape(s, 1, e)
